# baseline (speedup 1.0000x reference)
import functools

import jax
import jax.numpy as jnp
from jax import lax
from jax.experimental import pallas as pl
from jax.experimental.pallas import tpu as pltpu
from jax.experimental.pallas import tpu_sc as plsc

N_TOK = 256
N_NODE = 10000
N_TOT = N_TOK + N_NODE
E = 32768
D = 64
D_TOK = 768
N_REL = 40

NC = 2
NS = 16
NW = NC * NS

f32 = jnp.float32


def _sc_mesh():
    return plsc.VectorSubcoreMesh(
        core_axis_name="c", subcore_axis_name="s", num_cores=NC, num_subcores=NS
    )


@functools.lru_cache(maxsize=None)
def _make_gather(V, B, CH):
    b_per_w = B // NW
    n_ch = b_per_w // CH
    assert b_per_w % CH == 0

    def body(table_hbm, idx_hbm, out_hbm, idx_v, rows_v, sem):
        wid = lax.axis_index("s") * NC + lax.axis_index("c")
        pltpu.sync_copy(idx_hbm.at[wid], idx_v)
        descs = [
            pltpu.async_copy(table_hbm.at[idx_v.at[j]], rows_v.at[pl.ds(j * CH, CH)], sem)
            for j in range(n_ch)
        ]
        for dsc in descs:
            dsc.wait()
        pltpu.sync_copy(rows_v, out_hbm.at[pl.ds(wid * b_per_w, b_per_w)])

    return pl.kernel(
        body,
        out_type=jax.ShapeDtypeStruct((B, D), f32),
        mesh=_sc_mesh(),
        compiler_params=pltpu.CompilerParams(use_tc_tiling_on_sc=False),
        scratch_types=[
            pltpu.VMEM((n_ch, CH), jnp.int32),
            pltpu.VMEM((b_per_w, D), f32),
            pltpu.SemaphoreType.DMA,
        ],
    )


NT_PAD = 10496
R_PER_T = NT_PAD // NS


@functools.lru_cache(maxsize=None)
def _make_scatter():
    e_per_w = E // NW
    CH = 128
    n_ch = e_per_w // CH

    def body(msg_hbm, dst_hbm, zero_hbm, out_hbm, idx_v, rows_v, accum, sem):
        cid = lax.axis_index("c")
        sid = lax.axis_index("s")
        wid = sid * NC + cid
        pltpu.sync_copy(
            zero_hbm.at[pl.ds(sid * R_PER_T, R_PER_T)],
            accum.at[pl.ds(sid * R_PER_T, R_PER_T)],
        )
        plsc.subcore_barrier()
        pltpu.sync_copy(dst_hbm.at[wid], idx_v)
        pltpu.async_copy(msg_hbm.at[pl.ds(wid * e_per_w, e_per_w)], rows_v, sem).wait()
        for j in range(n_ch):
            pltpu.sync_copy(
                rows_v.at[pl.ds(j * CH, CH)], accum.at[idx_v.at[j]], add=True
            )
        plsc.subcore_barrier()
        pltpu.sync_copy(
            accum.at[pl.ds(sid * R_PER_T, R_PER_T)],
            out_hbm.at[cid].at[pl.ds(sid * R_PER_T, R_PER_T)],
        )

    return pl.kernel(
        body,
        out_type=jax.ShapeDtypeStruct((NC, NT_PAD, D), f32),
        mesh=_sc_mesh(),
        compiler_params=pltpu.CompilerParams(use_tc_tiling_on_sc=False),
        scratch_types=[
            pltpu.VMEM((n_ch, CH), jnp.int32),
            pltpu.VMEM((e_per_w, D), f32),
            pltpu.VMEM_SHARED((NT_PAD, D), f32),
            pltpu.SemaphoreType.DMA,
        ],
    )


def _te_body(tok_ref, w1_ref, b1_ref, o_ref):
    o_ref[...] = (
        jnp.dot(tok_ref[...], w1_ref[...], preferred_element_type=f32) + b1_ref[...]
    )


def _te_prep(tok, W1, b1):
    return pl.pallas_call(
        _te_body,
        out_shape=jax.ShapeDtypeStruct((N_TOK, D), f32),
    )(tok, W1, b1)


BE = 512


def _msg_body(ea_ref, xj_ref, ee_ref, wen_ref, ben_ref, o_ref):
    emb = jnp.dot(ea_ref[...], ee_ref[...], preferred_element_type=f32)
    w = jnp.maximum(
        jnp.dot(emb, wen_ref[...], preferred_element_type=f32) + ben_ref[...], 0.0
    )
    xj = xj_ref[...]
    acc = jnp.zeros((BE, D), f32)
    for d in range(D):
        acc = acc + xj[:, d : d + 1] * w[:, d * D : (d + 1) * D]
    o_ref[...] = acc


def _msg(edge_attr, x_j, edge_emb, W_en, b_en):
    return pl.pallas_call(
        _msg_body,
        grid=(E // BE,),
        in_specs=[
            pl.BlockSpec((BE, N_REL), lambda i: (i, 0)),
            pl.BlockSpec((BE, D), lambda i: (i, 0)),
            pl.BlockSpec((N_REL, D), lambda i: (0, 0)),
            pl.BlockSpec((D, D * D), lambda i: (0, 0)),
            pl.BlockSpec((D * D,), lambda i: (0,)),
        ],
        out_specs=pl.BlockSpec((BE, D), lambda i: (i, 0)),
        out_shape=jax.ShapeDtypeStruct((E, D), f32),
    )(edge_attr, x_j, edge_emb, W_en, b_en)


def _combine_body(p0_ref, p1_ref, x_ref, root_ref, bias_ref, o_ref, *, do_relu):
    v = (
        p0_ref[...]
        + p1_ref[...]
        + jnp.dot(x_ref[...], root_ref[...], preferred_element_type=f32)
        + bias_ref[...]
    )
    o_ref[...] = jnp.maximum(v, 0.0) if do_relu else v


def _combine(p0, p1, x, root, bias, do_relu):
    return pl.pallas_call(
        functools.partial(_combine_body, do_relu=do_relu),
        out_shape=jax.ShapeDtypeStruct((N_TOT, D), f32),
    )(p0, p1, x, root, bias)


BN = 5128


def _epiA_body(fx_ref, w2_ref, b2_ref, wnt_ref, bnt_ref, lab_ref, o_ref, nt_ref):
    i = pl.program_id(0)
    fx = fx_ref[...]
    o_ref[...] = jnp.dot(fx, w2_ref[...], preferred_element_type=f32) + b2_ref[...]
    logits = jnp.dot(fx, wnt_ref[...], preferred_element_type=f32) + bnt_ref[...]
    m = jnp.max(logits, axis=1, keepdims=True)
    lse = m + jnp.log(jnp.sum(jnp.exp(logits - m), axis=1, keepdims=True))
    logp = logits - lse
    oh = (lab_ref[...] == lax.broadcasted_iota(jnp.int32, (1, 3), 1)).astype(f32)
    picked = jnp.sum(logp * oh, keepdims=True)

    @pl.when(i == 0)
    def _():
        nt_ref[...] = jnp.zeros((1, 1), f32)

    nt_ref[...] += -picked / N_TOT


def _epiA(fx, W2, b2, W_nt, b_nt, labels2d):
    return pl.pallas_call(
        _epiA_body,
        grid=(N_TOT // BN,),
        in_specs=[
            pl.BlockSpec((BN, D), lambda i: (i, 0)),
            pl.BlockSpec((D, D_TOK), lambda i: (0, 0)),
            pl.BlockSpec((D_TOK,), lambda i: (0,)),
            pl.BlockSpec((D, 3), lambda i: (0, 0)),
            pl.BlockSpec((3,), lambda i: (0,)),
            pl.BlockSpec((BN, 1), lambda i: (i, 0)),
        ],
        out_specs=[
            pl.BlockSpec((BN, D_TOK), lambda i: (i, 0)),
            pl.BlockSpec((1, 1), lambda i: (0, 0)),
        ],
        out_shape=[
            jax.ShapeDtypeStruct((N_TOT, D_TOK), f32),
            jax.ShapeDtypeStruct((1, 1), f32),
        ],
    )(fx, W2, b2, W_nt, b_nt, labels2d)


BEL = 2048


def _epiB_body(ea_ref, fs_ref, fd_ref, ee_ref, kge_ref):
    i = pl.program_id(0)
    ea = ea_ref[...]
    eemb = jnp.dot(ea, ee_ref[...], preferred_element_type=f32)
    dlt = fs_ref[...] + eemb - fd_ref[...]
    mask = (jnp.sum(ea[:, N_REL - 3 :], axis=1) == 0.0).astype(f32)[:, None]
    s = jnp.sum(dlt * dlt * mask, keepdims=True)

    @pl.when(i == 0)
    def _():
        kge_ref[...] = jnp.zeros((1, 1), f32)

    kge_ref[...] += s / (E * D)


def _epiB(edge_attr, fs, fd, edge_emb):
    return pl.pallas_call(
        _epiB_body,
        grid=(E // BEL,),
        in_specs=[
            pl.BlockSpec((BEL, N_REL), lambda i: (i, 0)),
            pl.BlockSpec((BEL, D), lambda i: (i, 0)),
            pl.BlockSpec((BEL, D), lambda i: (i, 0)),
            pl.BlockSpec((N_REL, D), lambda i: (0, 0)),
        ],
        out_specs=pl.BlockSpec((1, 1), lambda i: (0, 0)),
        out_shape=jax.ShapeDtypeStruct((1, 1), f32),
    )(edge_attr, fs, fd, edge_emb)


def kernel(node_ids, edge_index, edge_attr, token_embeddings, node_type_labels,
           num_recognized_tokens, mask_out_rate,
           kg_emb, edge_emb, W_en, b_en, W1, b1, W2, b2,
           root1, bias1, root2, bias2, W_nt, b_nt):
    node_ids = node_ids.astype(jnp.int32)
    src = edge_index[0].astype(jnp.int32)
    dst = edge_index[1].astype(jnp.int32)

    B_NE = 12288
    nid_pad = jnp.concatenate(
        [node_ids, jnp.zeros((B_NE - N_NODE,), jnp.int32)]
    ).reshape(NW, -1, 128)
    ne = _make_gather(100000, B_NE, 128)(kg_emb, nid_pad)[:N_NODE]
    te = _te_prep(token_embeddings, W1, b1)
    x0 = jnp.concatenate([te, ne], axis=0)

    src2 = src.reshape(NW, -1, 128)
    dst2 = dst.reshape(NW, -1, 128)
    zero_init = jnp.zeros((NT_PAD, D), f32)
    gather_x = _make_gather(N_TOT, E, 128)
    scatter = _make_scatter()

    xj1 = gather_x(x0, src2)
    msg1 = _msg(edge_attr, xj1, edge_emb, W_en, b_en)
    p1 = scatter(msg1, dst2, zero_init)
    x1 = _combine(p1[0, :N_TOT], p1[1, :N_TOT], x0, root1, bias1, True)

    xj2 = gather_x(x1, src2)
    msg2 = _msg(edge_attr, xj2, edge_emb, W_en, b_en)
    p2 = scatter(msg2, dst2, zero_init)
    fx = _combine(p2[0, :N_TOT], p2[1, :N_TOT], x1, root2, bias2, False)

    fs = gather_x(fx, src2)
    fd = gather_x(fx, dst2)
    final_outputs, nt = _epiA(
        fx, W2, b2, W_nt, b_nt, node_type_labels.astype(jnp.int32).reshape(-1, 1)
    )
    kge = _epiB(edge_attr, fs, fd, edge_emb)
    return (
        final_outputs,
        kge.reshape(()).astype(f32),
        nt.reshape(()).astype(f32),
        jnp.float32(0.0),
    )
